# baseline (speedup 1.0000x reference)
import functools

import jax
import jax.numpy as jnp
from jax import lax
from jax.experimental import pallas as pl
from jax.experimental.pallas import tpu as pltpu
from jax.experimental.pallas import tpu_sc as plsc

D_MODEL = 1024
D_HIDDEN = 256
N_SHARED = 2
N_ROUTED = 16
TOP_K = 2
T_TOKENS = 2048
N_PAIRS = T_TOKENS * TOP_K
B_SORT = 128
L_PAD = N_PAIRS + N_ROUTED * B_SORT
NB = L_PAD // B_SORT
T_BLOCK = 256
N_TB = T_TOKENS // T_BLOCK

N_WORKER = 32
PW = N_PAIRS // N_WORKER
N_CHUNK = PW // 16


def _tc_shared_body(x_ref, Ws1_ref, bs1_ref, Ws2_ref, bs2_ref, sh_ref):
    xb = x_ref[...]
    xb16 = xb.astype(jnp.bfloat16)
    acc = jnp.zeros_like(xb)
    for s in range(N_SHARED):
        h = jnp.maximum(
            jnp.dot(xb16, Ws1_ref[s].astype(jnp.bfloat16),
                    preferred_element_type=jnp.float32)
            + bs1_ref[s][None, :], 0.0)
        acc = acc + jnp.dot(h.astype(jnp.bfloat16),
                            Ws2_ref[s].astype(jnp.bfloat16),
                            preferred_element_type=jnp.float32) \
            + bs2_ref[s][None, :]
    sh_ref[...] = acc * (1.0 / N_SHARED)


def _tc_shared(flat, Ws1, bs1, Ws2, bs2, interpret=False):
    return pl.pallas_call(
        _tc_shared_body,
        grid=(N_TB,),
        in_specs=[
            pl.BlockSpec((T_BLOCK, D_MODEL), lambda i: (i, 0)),
            _full(Ws1.shape), _full(bs1.shape),
            _full(Ws2.shape), _full(bs2.shape),
        ],
        out_specs=pl.BlockSpec((T_BLOCK, D_MODEL), lambda i: (i, 0)),
        out_shape=jax.ShapeDtypeStruct((T_TOKENS, D_MODEL), jnp.float32),
        interpret=interpret,
    )(flat, Ws1, bs1, Ws2, bs2)


def _tc1_body(x_ref, Wg_ref, bg_ref,
              eidx_ref, ew_ref, rtot_ref, base_ref, bexp_ref,
              carry_ref):
    i = pl.program_id(0)
    xb = x_ref[...]
    logits = jnp.dot(xb, Wg_ref[...], preferred_element_type=jnp.float32) \
        + bg_ref[...][None, :]
    iota = lax.broadcasted_iota(jnp.int32, logits.shape, 1)
    m1 = jnp.max(logits, axis=-1, keepdims=True)
    i1 = jnp.min(jnp.where(logits == m1, iota, N_ROUTED), axis=-1, keepdims=True)
    masked = jnp.where(iota == i1, -jnp.inf, logits)
    m2 = jnp.max(masked, axis=-1, keepdims=True)
    i2 = jnp.min(jnp.where(masked == m2, iota, N_ROUTED), axis=-1, keepdims=True)
    w1 = 1.0 / (1.0 + jnp.exp(m2 - m1))
    eidx_ref[...] = jnp.concatenate([i1, i2], axis=1)
    ew_ref[...] = jnp.concatenate([w1, 1.0 - w1], axis=1)

    @pl.when(i == 0)
    def _():
        carry_ref[...] = jnp.zeros((1, N_ROUTED), jnp.float32)

    m0 = (iota == i1).astype(jnp.float32)
    m1h = (iota == i2).astype(jnp.float32)
    r = lax.broadcasted_iota(jnp.int32, (T_BLOCK, T_BLOCK), 0)
    c = lax.broadcasted_iota(jnp.int32, (T_BLOCK, T_BLOCK), 1)
    tril = (r > c).astype(jnp.float32)
    c0 = jnp.dot(tril, m0, preferred_element_type=jnp.float32)
    c1 = jnp.dot(tril, m1h, preferred_element_type=jnp.float32)
    carry = carry_ref[...]
    both = c0 + c1 + carry
    r0 = jnp.sum(both * m0, axis=1, keepdims=True)
    r1 = jnp.sum(both * m1h, axis=1, keepdims=True)
    rtot_ref[...] = jnp.concatenate([r0, r1], axis=1).astype(jnp.int32)
    new_carry = carry + jnp.sum(m0 + m1h, axis=0, keepdims=True)
    carry_ref[...] = new_carry

    @pl.when(i == N_TB - 1)
    def _():
        n = new_carry
        padded = jnp.floor((n + (B_SORT - 1)) * (1.0 / B_SORT)) * B_SORT
        e_r = lax.broadcasted_iota(jnp.int32, (N_ROUTED, N_ROUTED), 0)
        e_c = lax.broadcasted_iota(jnp.int32, (N_ROUTED, N_ROUTED), 1)
        incl = (e_r <= e_c).astype(jnp.float32)
        off_incl = jnp.dot(padded, incl, preferred_element_type=jnp.float32)
        base_ref[...] = (off_incl - padded).astype(jnp.int32)
        thr = lax.broadcasted_iota(jnp.int32, (1, NB), 1).astype(jnp.float32) \
            * float(B_SORT)
        cnt = jnp.zeros((1, NB), jnp.float32)
        for e in range(N_ROUTED):
            cnt = cnt + (off_incl[:, e:e + 1] <= thr).astype(jnp.float32)
        bexp_ref[...] = jnp.minimum(cnt, float(N_ROUTED - 1)).astype(jnp.int32)


def _full(shape):
    return pl.BlockSpec(shape, lambda i: tuple(0 for _ in shape))


def _tc1(flat, Wg, bg, interpret=False):
    grid = (N_TB,)
    return pl.pallas_call(
        _tc1_body,
        grid=grid,
        in_specs=[
            pl.BlockSpec((T_BLOCK, D_MODEL), lambda i: (i, 0)),
            _full(Wg.shape), _full(bg.shape),
        ],
        out_specs=[
            pl.BlockSpec((T_BLOCK, TOP_K), lambda i: (i, 0)),
            pl.BlockSpec((T_BLOCK, TOP_K), lambda i: (i, 0)),
            pl.BlockSpec((T_BLOCK, TOP_K), lambda i: (i, 0)),
            pl.BlockSpec((1, N_ROUTED), lambda i: (0, 0)),
            pl.BlockSpec((1, NB), lambda i: (0, 0)),
        ],
        out_shape=[
            jax.ShapeDtypeStruct((T_TOKENS, TOP_K), jnp.int32),
            jax.ShapeDtypeStruct((T_TOKENS, TOP_K), jnp.float32),
            jax.ShapeDtypeStruct((T_TOKENS, TOP_K), jnp.int32),
            jax.ShapeDtypeStruct((1, N_ROUTED), jnp.int32),
            jax.ShapeDtypeStruct((1, NB), jnp.int32),
        ],
        scratch_shapes=[pltpu.VMEM((1, N_ROUTED), jnp.float32)],
        interpret=interpret,
    )(flat, Wg, bg)


def _worker_id():
    return lax.axis_index("c") * (N_WORKER // 2) + lax.axis_index("s")


def _sc_move_body(eflat_hbm, rtot_hbm, basew_hbm, x_hbm,
                  xs_hbm, pos_hbm, e_buf, rtot_v, base_v, pos_v,
                  rows_a, rows_b, gsem, ssem):
    wid = _worker_id()
    iota = lax.broadcasted_iota(jnp.int32, (16,), 0)

    pltpu.sync_copy(eflat_hbm.at[pl.ds(wid * PW, PW)], e_buf)
    pltpu.sync_copy(rtot_hbm.at[pl.ds(wid * PW, PW)], rtot_v)
    pltpu.sync_copy(basew_hbm, base_v)

    poss = []
    for c in range(N_CHUNK):
        ev = e_buf[pl.ds(c * 16, 16)]
        rt = rtot_v[pl.ds(c * 16, 16)]
        posc = plsc.load_gather(base_v, [ev]) + rt
        pos_v[c, pl.ds(0, 16)] = posc
        poss.append(posc)
    pltpu.sync_copy(pos_v, pos_hbm.at[wid])

    bufs = (rows_a, rows_b)
    gd = [None] * N_CHUNK
    sd = [None] * N_CHUNK
    tok0 = (wid * PW + iota) >> 1
    gd[0] = pltpu.async_copy(x_hbm.at[tok0], bufs[0], gsem)
    for c in range(N_CHUNK):
        buf = bufs[c % 2]
        if c >= 1:
            sd[c - 1].wait()
        gd[c].wait()
        sd[c] = pltpu.async_copy(buf, xs_hbm.at[poss[c]], ssem)
        if c + 1 < N_CHUNK:
            tok = (wid * PW + (c + 1) * 16 + iota) >> 1
            gd[c + 1] = pltpu.async_copy(x_hbm.at[tok], bufs[(c + 1) % 2], gsem)
    sd[N_CHUNK - 1].wait()


def _sc_combine_body(pos_hbm, ew_hbm, sh_hbm, y_hbm, out_hbm,
                     posv, wv, yr_a, yr_b, sh_a, sh_b, outv,
                     ysem, hsem, osem):
    wid = _worker_id()
    iota = lax.broadcasted_iota(jnp.int32, (16,), 0)

    pltpu.sync_copy(pos_hbm.at[wid], posv)
    pltpu.sync_copy(ew_hbm.at[pl.ds(wid * PW, PW)], wv)

    ybufs = (yr_a, yr_b)
    sbufs = (sh_a, sh_b)
    gd = [None] * N_CHUNK
    hd = [None] * N_CHUNK
    od = [None] * N_CHUNK

    def _issue(c):
        pr = posv[c, pl.ds(0, 16)]
        gd[c] = pltpu.async_copy(y_hbm.at[pr], ybufs[c % 2], ysem)
        tb = wid * (PW // 2) + c * 8
        hd[c] = pltpu.async_copy(sh_hbm.at[pl.ds(tb, 8)], sbufs[c % 2], hsem)

    _issue(0)
    for c in range(N_CHUNK):
        yrows = ybufs[c % 2]
        shv = sbufs[c % 2]
        gd[c].wait()
        hd[c].wait()
        wreg = wv[pl.ds(c * 16, 16)]
        ws = []
        for j in range(8):
            ws.append(jnp.sum(jnp.where(iota == 2 * j, wreg, 0.0)))
            ws.append(jnp.sum(jnp.where(iota == 2 * j + 1, wreg, 0.0)))
        if c >= 1:
            od[c - 1].wait()

        def _col(col, _, ws=ws, yrows=yrows, shv=shv):
            s = pl.ds(col * 16, 16)
            for j in range(8):
                outv[j, s] = shv[j, s] + ws[2 * j] * yrows[2 * j, s] \
                    + ws[2 * j + 1] * yrows[2 * j + 1, s]
            return 0

        lax.fori_loop(0, D_MODEL // 16, _col, 0)
        tb = wid * (PW // 2) + c * 8
        od[c] = pltpu.async_copy(outv, out_hbm.at[pl.ds(tb, 8)], osem)
        if c + 1 < N_CHUNK:
            _issue(c + 1)
    od[N_CHUNK - 1].wait()


@functools.cache
def _sc_kernels():
    mesh = plsc.VectorSubcoreMesh(core_axis_name="c", subcore_axis_name="s")
    sc_params = pltpu.CompilerParams(needs_layout_passes=False)
    move = pl.kernel(
        _sc_move_body,
        out_type=(
            jax.ShapeDtypeStruct((L_PAD, D_MODEL), jnp.float32),
            jax.ShapeDtypeStruct((N_WORKER, N_CHUNK, 16), jnp.int32),
        ),
        mesh=mesh,
        compiler_params=sc_params,
        scratch_types=(
            pltpu.VMEM((PW,), jnp.int32),
            pltpu.VMEM((PW,), jnp.int32),
            pltpu.VMEM((16,), jnp.int32),
            pltpu.VMEM((N_CHUNK, 16), jnp.int32),
            pltpu.VMEM((16, D_MODEL), jnp.float32),
            pltpu.VMEM((16, D_MODEL), jnp.float32),
            pltpu.SemaphoreType.DMA,
            pltpu.SemaphoreType.DMA,
        ),
    )
    combine = pl.kernel(
        _sc_combine_body,
        out_type=jax.ShapeDtypeStruct((T_TOKENS, D_MODEL), jnp.float32),
        mesh=mesh,
        compiler_params=sc_params,
        scratch_types=(
            pltpu.VMEM((N_CHUNK, 16), jnp.int32),
            pltpu.VMEM((PW,), jnp.float32),
            pltpu.VMEM((16, D_MODEL), jnp.float32),
            pltpu.VMEM((16, D_MODEL), jnp.float32),
            pltpu.VMEM((8, D_MODEL), jnp.float32),
            pltpu.VMEM((8, D_MODEL), jnp.float32),
            pltpu.VMEM((8, D_MODEL), jnp.float32),
            pltpu.SemaphoreType.DMA,
            pltpu.SemaphoreType.DMA,
            pltpu.SemaphoreType.DMA,
        ),
    )
    return move, combine


def _gmm_body(bexp_ref, xs_ref, w1_ref, w2_ref, b1_ref, b2_ref, y_ref):
    xb = xs_ref[...].astype(jnp.bfloat16)
    h = jnp.maximum(
        jnp.dot(xb, w1_ref[0], preferred_element_type=jnp.float32)
        + b1_ref[0], 0.0)
    y_ref[...] = jnp.dot(h.astype(jnp.bfloat16), w2_ref[0],
                         preferred_element_type=jnp.float32) + b2_ref[0]


def _gmm(bexp, xs, Wr1, Wr2, br1r, br2r, interpret=False):
    grid_spec = pltpu.PrefetchScalarGridSpec(
        num_scalar_prefetch=1,
        grid=(NB,),
        in_specs=[
            pl.BlockSpec((B_SORT, D_MODEL), lambda i, b: (i, 0)),
            pl.BlockSpec((1, D_MODEL, D_HIDDEN), lambda i, b: (b[i], 0, 0)),
            pl.BlockSpec((1, D_HIDDEN, D_MODEL), lambda i, b: (b[i], 0, 0)),
            pl.BlockSpec((1, 1, D_HIDDEN), lambda i, b: (b[i], 0, 0)),
            pl.BlockSpec((1, 1, D_MODEL), lambda i, b: (b[i], 0, 0)),
        ],
        out_specs=pl.BlockSpec((B_SORT, D_MODEL), lambda i, b: (i, 0)),
    )
    return pl.pallas_call(
        _gmm_body,
        grid_spec=grid_spec,
        out_shape=jax.ShapeDtypeStruct((L_PAD, D_MODEL), jnp.float32),
        interpret=interpret,
    )(bexp, xs, Wr1, Wr2, br1r, br2r)


def kernel(x, Ws1, bs1, Ws2, bs2, Wr1, br1, Wr2, br2, Wg, bg):
    B, S, D = x.shape
    flat = x.reshape(T_TOKENS, D_MODEL)
    eidx, ew, rtot, base, bexp = _tc1(flat, Wg, bg)
    shared = _tc_shared(flat, Ws1, bs1, Ws2, bs2)
    eflat = eidx.reshape(N_PAIRS)
    ewflat = ew.reshape(N_PAIRS)
    move, combine = _sc_kernels()
    xs, pos = move(eflat, rtot.reshape(N_PAIRS), base.reshape(N_ROUTED), flat)
    y = _gmm(bexp.reshape(NB), xs,
             Wr1.astype(jnp.bfloat16), Wr2.astype(jnp.bfloat16),
             br1.reshape(N_ROUTED, 1, D_HIDDEN),
             br2.reshape(N_ROUTED, 1, D_MODEL))
    out = combine(pos, ewflat, shared, y)
    return out.reshape(B, S, D)


# speedup vs baseline: 1.1159x; 1.1159x over previous
import functools

import jax
import jax.numpy as jnp
from jax import lax
from jax.experimental import pallas as pl
from jax.experimental.pallas import tpu as pltpu
from jax.experimental.pallas import tpu_sc as plsc

D_MODEL = 1024
D_HIDDEN = 256
N_SHARED = 2
N_ROUTED = 16
TOP_K = 2
T_TOKENS = 2048
N_PAIRS = T_TOKENS * TOP_K
B_SORT = 128
L_PAD = N_PAIRS + N_ROUTED * B_SORT
NB = L_PAD // B_SORT
T_BLOCK = 256
N_TB = T_TOKENS // T_BLOCK

N_WORKER = 32
PW = N_PAIRS // N_WORKER
N_CHUNK = PW // 16


def _tc_shared_body(x_ref, Ws1_ref, bs1_ref, Ws2_ref, bs2_ref, sh_ref):
    xb = x_ref[...]
    xb16 = xb.astype(jnp.bfloat16)
    acc = jnp.zeros_like(xb)
    for s in range(N_SHARED):
        h = jnp.maximum(
            jnp.dot(xb16, Ws1_ref[s].astype(jnp.bfloat16),
                    preferred_element_type=jnp.float32)
            + bs1_ref[s][None, :], 0.0)
        acc = acc + jnp.dot(h.astype(jnp.bfloat16),
                            Ws2_ref[s].astype(jnp.bfloat16),
                            preferred_element_type=jnp.float32) \
            + bs2_ref[s][None, :]
    sh_ref[...] = acc * (1.0 / N_SHARED)


def _tc_shared(flat, Ws1, bs1, Ws2, bs2, interpret=False):
    return pl.pallas_call(
        _tc_shared_body,
        grid=(N_TB,),
        in_specs=[
            pl.BlockSpec((T_BLOCK, D_MODEL), lambda i: (i, 0)),
            _full(Ws1.shape), _full(bs1.shape),
            _full(Ws2.shape), _full(bs2.shape),
        ],
        out_specs=pl.BlockSpec((T_BLOCK, D_MODEL), lambda i: (i, 0)),
        out_shape=jax.ShapeDtypeStruct((T_TOKENS, D_MODEL), jnp.float32),
        interpret=interpret,
    )(flat, Ws1, bs1, Ws2, bs2)


def _tc1_body(x_ref, Wg_ref, bg_ref,
              eidx_ref, ew_ref, rtot_ref, base_ref, bexp_ref,
              carry_ref):
    i = pl.program_id(0)
    xb = x_ref[...]
    logits = jnp.dot(xb, Wg_ref[...], preferred_element_type=jnp.float32) \
        + bg_ref[...][None, :]
    iota = lax.broadcasted_iota(jnp.int32, logits.shape, 1)
    m1 = jnp.max(logits, axis=-1, keepdims=True)
    i1 = jnp.min(jnp.where(logits == m1, iota, N_ROUTED), axis=-1, keepdims=True)
    masked = jnp.where(iota == i1, -jnp.inf, logits)
    m2 = jnp.max(masked, axis=-1, keepdims=True)
    i2 = jnp.min(jnp.where(masked == m2, iota, N_ROUTED), axis=-1, keepdims=True)
    w1 = 1.0 / (1.0 + jnp.exp(m2 - m1))
    eidx_ref[...] = jnp.concatenate([i1, i2], axis=1)
    ew_ref[...] = jnp.concatenate([w1, 1.0 - w1], axis=1)

    @pl.when(i == 0)
    def _():
        carry_ref[...] = jnp.zeros((1, N_ROUTED), jnp.float32)

    m0 = (iota == i1).astype(jnp.float32)
    m1h = (iota == i2).astype(jnp.float32)
    r = lax.broadcasted_iota(jnp.int32, (T_BLOCK, T_BLOCK), 0)
    c = lax.broadcasted_iota(jnp.int32, (T_BLOCK, T_BLOCK), 1)
    tril = (r > c).astype(jnp.float32)
    c0 = jnp.dot(tril, m0, preferred_element_type=jnp.float32)
    c1 = jnp.dot(tril, m1h, preferred_element_type=jnp.float32)
    carry = carry_ref[...]
    both = c0 + c1 + carry
    r0 = jnp.sum(both * m0, axis=1, keepdims=True)
    r1 = jnp.sum(both * m1h, axis=1, keepdims=True)
    rtot_ref[...] = jnp.concatenate([r0, r1], axis=1).astype(jnp.int32)
    new_carry = carry + jnp.sum(m0 + m1h, axis=0, keepdims=True)
    carry_ref[...] = new_carry

    @pl.when(i == N_TB - 1)
    def _():
        n = new_carry
        padded = jnp.floor((n + (B_SORT - 1)) * (1.0 / B_SORT)) * B_SORT
        e_r = lax.broadcasted_iota(jnp.int32, (N_ROUTED, N_ROUTED), 0)
        e_c = lax.broadcasted_iota(jnp.int32, (N_ROUTED, N_ROUTED), 1)
        incl = (e_r <= e_c).astype(jnp.float32)
        off_incl = jnp.dot(padded, incl, preferred_element_type=jnp.float32)
        base_ref[...] = (off_incl - padded).astype(jnp.int32)
        thr = lax.broadcasted_iota(jnp.int32, (1, NB), 1).astype(jnp.float32) \
            * float(B_SORT)
        cnt = jnp.zeros((1, NB), jnp.float32)
        for e in range(N_ROUTED):
            cnt = cnt + (off_incl[:, e:e + 1] <= thr).astype(jnp.float32)
        bexp_ref[...] = jnp.minimum(cnt, float(N_ROUTED - 1)).astype(jnp.int32)


def _full(shape):
    return pl.BlockSpec(shape, lambda i: tuple(0 for _ in shape))


def _full2(shape):
    return pl.BlockSpec(shape, lambda i, b: tuple(0 for _ in shape))


def _tc1(flat, Wg, bg, interpret=False):
    grid = (N_TB,)
    return pl.pallas_call(
        _tc1_body,
        grid=grid,
        in_specs=[
            pl.BlockSpec((T_BLOCK, D_MODEL), lambda i: (i, 0)),
            _full(Wg.shape), _full(bg.shape),
        ],
        out_specs=[
            pl.BlockSpec((T_BLOCK, TOP_K), lambda i: (i, 0)),
            pl.BlockSpec((T_BLOCK, TOP_K), lambda i: (i, 0)),
            pl.BlockSpec((T_BLOCK, TOP_K), lambda i: (i, 0)),
            pl.BlockSpec((1, N_ROUTED), lambda i: (0, 0)),
            pl.BlockSpec((1, NB), lambda i: (0, 0)),
        ],
        out_shape=[
            jax.ShapeDtypeStruct((T_TOKENS, TOP_K), jnp.int32),
            jax.ShapeDtypeStruct((T_TOKENS, TOP_K), jnp.float32),
            jax.ShapeDtypeStruct((T_TOKENS, TOP_K), jnp.int32),
            jax.ShapeDtypeStruct((1, N_ROUTED), jnp.int32),
            jax.ShapeDtypeStruct((1, NB), jnp.int32),
        ],
        scratch_shapes=[pltpu.VMEM((1, N_ROUTED), jnp.float32)],
        interpret=interpret,
    )(flat, Wg, bg)


def _worker_id():
    return lax.axis_index("c") * (N_WORKER // 2) + lax.axis_index("s")


def _sc_move_body(eflat_hbm, rtot_hbm, basew_hbm, x_hbm,
                  xs_hbm, pos_hbm, e_buf, rtot_v, base_v, pos_v,
                  rows_a, rows_b, gsem, ssem):
    wid = _worker_id()
    iota = lax.broadcasted_iota(jnp.int32, (16,), 0)

    pltpu.sync_copy(eflat_hbm.at[pl.ds(wid * PW, PW)], e_buf)
    pltpu.sync_copy(rtot_hbm.at[pl.ds(wid * PW, PW)], rtot_v)
    pltpu.sync_copy(basew_hbm, base_v)

    poss = []
    for c in range(N_CHUNK):
        ev = e_buf[pl.ds(c * 16, 16)]
        rt = rtot_v[pl.ds(c * 16, 16)]
        posc = plsc.load_gather(base_v, [ev]) + rt
        pos_v[c, pl.ds(0, 16)] = posc
        poss.append(posc)
    pltpu.sync_copy(pos_v, pos_hbm.at[wid])

    bufs = (rows_a, rows_b)
    gd = [None] * N_CHUNK
    sd = [None] * N_CHUNK
    tok0 = (wid * PW + iota) >> 1
    gd[0] = pltpu.async_copy(x_hbm.at[tok0], bufs[0], gsem)
    for c in range(N_CHUNK):
        buf = bufs[c % 2]
        if c >= 1:
            sd[c - 1].wait()
        gd[c].wait()
        sd[c] = pltpu.async_copy(buf, xs_hbm.at[poss[c]], ssem)
        if c + 1 < N_CHUNK:
            tok = (wid * PW + (c + 1) * 16 + iota) >> 1
            gd[c + 1] = pltpu.async_copy(x_hbm.at[tok], bufs[(c + 1) % 2], gsem)
    sd[N_CHUNK - 1].wait()


def _sc_combine_body(pos_hbm, ew_hbm, sh_hbm, y_hbm, out_hbm,
                     posv, wv, yr_a, yr_b, sh_a, sh_b, outv,
                     ysem, hsem, osem):
    wid = _worker_id()
    iota = lax.broadcasted_iota(jnp.int32, (16,), 0)

    pltpu.sync_copy(pos_hbm.at[wid], posv)
    pltpu.sync_copy(ew_hbm.at[pl.ds(wid * PW, PW)], wv)

    ybufs = (yr_a, yr_b)
    sbufs = (sh_a, sh_b)
    gd = [None] * N_CHUNK
    hd = [None] * N_CHUNK
    od = [None] * N_CHUNK

    def _issue(c):
        pr = posv[c, pl.ds(0, 16)]
        gd[c] = pltpu.async_copy(y_hbm.at[pr], ybufs[c % 2], ysem)
        tb = wid * (PW // 2) + c * 8
        hd[c] = pltpu.async_copy(sh_hbm.at[pl.ds(tb, 8)], sbufs[c % 2], hsem)

    _issue(0)
    for c in range(N_CHUNK):
        yrows = ybufs[c % 2]
        shv = sbufs[c % 2]
        gd[c].wait()
        hd[c].wait()
        wreg = wv[pl.ds(c * 16, 16)]
        ws = []
        for j in range(8):
            ws.append(jnp.sum(jnp.where(iota == 2 * j, wreg, 0.0)))
            ws.append(jnp.sum(jnp.where(iota == 2 * j + 1, wreg, 0.0)))
        if c >= 1:
            od[c - 1].wait()

        def _col(col, _, ws=ws, yrows=yrows, shv=shv):
            s = pl.ds(col * 16, 16)
            for j in range(8):
                outv[j, s] = shv[j, s] + ws[2 * j] * yrows[2 * j, s] \
                    + ws[2 * j + 1] * yrows[2 * j + 1, s]
            return 0

        lax.fori_loop(0, D_MODEL // 16, _col, 0)
        tb = wid * (PW // 2) + c * 8
        od[c] = pltpu.async_copy(outv, out_hbm.at[pl.ds(tb, 8)], osem)
        if c + 1 < N_CHUNK:
            _issue(c + 1)
    od[N_CHUNK - 1].wait()


@functools.cache
def _sc_kernels():
    mesh = plsc.VectorSubcoreMesh(core_axis_name="c", subcore_axis_name="s")
    sc_params = pltpu.CompilerParams(needs_layout_passes=False)
    move = pl.kernel(
        _sc_move_body,
        out_type=(
            jax.ShapeDtypeStruct((L_PAD, D_MODEL), jnp.float32),
            jax.ShapeDtypeStruct((N_WORKER, N_CHUNK, 16), jnp.int32),
        ),
        mesh=mesh,
        compiler_params=sc_params,
        scratch_types=(
            pltpu.VMEM((PW,), jnp.int32),
            pltpu.VMEM((PW,), jnp.int32),
            pltpu.VMEM((16,), jnp.int32),
            pltpu.VMEM((N_CHUNK, 16), jnp.int32),
            pltpu.VMEM((16, D_MODEL), jnp.float32),
            pltpu.VMEM((16, D_MODEL), jnp.float32),
            pltpu.SemaphoreType.DMA,
            pltpu.SemaphoreType.DMA,
        ),
    )
    combine = pl.kernel(
        _sc_combine_body,
        out_type=jax.ShapeDtypeStruct((T_TOKENS, D_MODEL), jnp.float32),
        mesh=mesh,
        compiler_params=sc_params,
        scratch_types=(
            pltpu.VMEM((N_CHUNK, 16), jnp.int32),
            pltpu.VMEM((PW,), jnp.float32),
            pltpu.VMEM((16, D_MODEL), jnp.float32),
            pltpu.VMEM((16, D_MODEL), jnp.float32),
            pltpu.VMEM((8, D_MODEL), jnp.float32),
            pltpu.VMEM((8, D_MODEL), jnp.float32),
            pltpu.VMEM((8, D_MODEL), jnp.float32),
            pltpu.SemaphoreType.DMA,
            pltpu.SemaphoreType.DMA,
            pltpu.SemaphoreType.DMA,
        ),
    )
    return move, combine


def _gmm_body(bexp_ref, xs_ref, w1_ref, w2_ref, b1_ref, b2_ref, y_ref):
    be = bexp_ref[pl.program_id(0)]
    xb = xs_ref[...].astype(jnp.bfloat16)
    w1 = w1_ref[be]
    w2 = w2_ref[be]
    h = jnp.maximum(
        jnp.dot(xb, w1.astype(jnp.bfloat16), preferred_element_type=jnp.float32)
        + b1_ref[be], 0.0)
    y_ref[...] = jnp.dot(h.astype(jnp.bfloat16), w2.astype(jnp.bfloat16),
                         preferred_element_type=jnp.float32) + b2_ref[be]


def _gmm(bexp, xs, Wr1, Wr2, br1r, br2r, interpret=False):
    grid_spec = pltpu.PrefetchScalarGridSpec(
        num_scalar_prefetch=1,
        grid=(NB,),
        in_specs=[
            pl.BlockSpec((B_SORT, D_MODEL), lambda i, b: (i, 0)),
            _full2(Wr1.shape), _full2(Wr2.shape),
            _full2(br1r.shape), _full2(br2r.shape),
        ],
        out_specs=pl.BlockSpec((B_SORT, D_MODEL), lambda i, b: (i, 0)),
    )
    return pl.pallas_call(
        _gmm_body,
        grid_spec=grid_spec,
        out_shape=jax.ShapeDtypeStruct((L_PAD, D_MODEL), jnp.float32),
        interpret=interpret,
    )(bexp, xs, Wr1, Wr2, br1r, br2r)


def kernel(x, Ws1, bs1, Ws2, bs2, Wr1, br1, Wr2, br2, Wg, bg):
    B, S, D = x.shape
    flat = x.reshape(T_TOKENS, D_MODEL)
    eidx, ew, rtot, base, bexp = _tc1(flat, Wg, bg)
    shared = _tc_shared(flat, Ws1, bs1, Ws2, bs2)
    eflat = eidx.reshape(N_PAIRS)
    ewflat = ew.reshape(N_PAIRS)
    move, combine = _sc_kernels()
    xs, pos = move(eflat, rtot.reshape(N_PAIRS), base.reshape(N_ROUTED), flat)
    y = _gmm(bexp.reshape(NB), xs, Wr1, Wr2,
             br1.reshape(N_ROUTED, 1, D_HIDDEN),
             br2.reshape(N_ROUTED, 1, D_MODEL))
    out = combine(pos, ewflat, shared, y)
    return out.reshape(B, S, D)


# speedup vs baseline: 1.1497x; 1.0302x over previous
import functools

import jax
import jax.numpy as jnp
from jax import lax
from jax.experimental import pallas as pl
from jax.experimental.pallas import tpu as pltpu
from jax.experimental.pallas import tpu_sc as plsc

D_MODEL = 1024
D_HIDDEN = 256
N_SHARED = 2
N_ROUTED = 16
TOP_K = 2
T_TOKENS = 2048
N_PAIRS = T_TOKENS * TOP_K
B_SORT = 256
L_PAD = N_PAIRS + N_ROUTED * B_SORT
NB = L_PAD // B_SORT
T_BLOCK = 256
N_TB = T_TOKENS // T_BLOCK

N_WORKER = 32
PW = N_PAIRS // N_WORKER
N_CHUNK = PW // 16


def _tc_shared_body(x_ref, Ws1_ref, bs1_ref, Ws2_ref, bs2_ref, sh_ref):
    xb = x_ref[...]
    xb16 = xb.astype(jnp.bfloat16)
    acc = jnp.zeros_like(xb)
    for s in range(N_SHARED):
        h = jnp.maximum(
            jnp.dot(xb16, Ws1_ref[s].astype(jnp.bfloat16),
                    preferred_element_type=jnp.float32)
            + bs1_ref[s][None, :], 0.0)
        acc = acc + jnp.dot(h.astype(jnp.bfloat16),
                            Ws2_ref[s].astype(jnp.bfloat16),
                            preferred_element_type=jnp.float32) \
            + bs2_ref[s][None, :]
    sh_ref[...] = acc * (1.0 / N_SHARED)


def _tc_shared(flat, Ws1, bs1, Ws2, bs2, interpret=False):
    return pl.pallas_call(
        _tc_shared_body,
        grid=(N_TB,),
        in_specs=[
            pl.BlockSpec((T_BLOCK, D_MODEL), lambda i: (i, 0)),
            _full(Ws1.shape), _full(bs1.shape),
            _full(Ws2.shape), _full(bs2.shape),
        ],
        out_specs=pl.BlockSpec((T_BLOCK, D_MODEL), lambda i: (i, 0)),
        out_shape=jax.ShapeDtypeStruct((T_TOKENS, D_MODEL), jnp.float32),
        interpret=interpret,
    )(flat, Ws1, bs1, Ws2, bs2)


def _tc1_body(x_ref, Wg_ref, bg_ref,
              eidx_ref, ew_ref, rtot_ref, base_ref, bexp_ref,
              carry_ref):
    i = pl.program_id(0)
    xb = x_ref[...]
    logits = jnp.dot(xb, Wg_ref[...], preferred_element_type=jnp.float32) \
        + bg_ref[...][None, :]
    iota = lax.broadcasted_iota(jnp.int32, logits.shape, 1)
    m1 = jnp.max(logits, axis=-1, keepdims=True)
    i1 = jnp.min(jnp.where(logits == m1, iota, N_ROUTED), axis=-1, keepdims=True)
    masked = jnp.where(iota == i1, -jnp.inf, logits)
    m2 = jnp.max(masked, axis=-1, keepdims=True)
    i2 = jnp.min(jnp.where(masked == m2, iota, N_ROUTED), axis=-1, keepdims=True)
    w1 = 1.0 / (1.0 + jnp.exp(m2 - m1))
    eidx_ref[...] = jnp.concatenate([i1, i2], axis=1)
    ew_ref[...] = jnp.concatenate([w1, 1.0 - w1], axis=1)

    @pl.when(i == 0)
    def _():
        carry_ref[...] = jnp.zeros((1, N_ROUTED), jnp.float32)

    m0 = (iota == i1).astype(jnp.float32)
    m1h = (iota == i2).astype(jnp.float32)
    r = lax.broadcasted_iota(jnp.int32, (T_BLOCK, T_BLOCK), 0)
    c = lax.broadcasted_iota(jnp.int32, (T_BLOCK, T_BLOCK), 1)
    tril = (r > c).astype(jnp.float32)
    c0 = jnp.dot(tril, m0, preferred_element_type=jnp.float32)
    c1 = jnp.dot(tril, m1h, preferred_element_type=jnp.float32)
    carry = carry_ref[...]
    both = c0 + c1 + carry
    r0 = jnp.sum(both * m0, axis=1, keepdims=True)
    r1 = jnp.sum(both * m1h, axis=1, keepdims=True)
    rtot_ref[...] = jnp.concatenate([r0, r1], axis=1).astype(jnp.int32)
    new_carry = carry + jnp.sum(m0 + m1h, axis=0, keepdims=True)
    carry_ref[...] = new_carry

    @pl.when(i == N_TB - 1)
    def _():
        n = new_carry
        padded = jnp.floor((n + (B_SORT - 1)) * (1.0 / B_SORT)) * B_SORT
        e_r = lax.broadcasted_iota(jnp.int32, (N_ROUTED, N_ROUTED), 0)
        e_c = lax.broadcasted_iota(jnp.int32, (N_ROUTED, N_ROUTED), 1)
        incl = (e_r <= e_c).astype(jnp.float32)
        off_incl = jnp.dot(padded, incl, preferred_element_type=jnp.float32)
        base_ref[...] = (off_incl - padded).astype(jnp.int32)
        thr = lax.broadcasted_iota(jnp.int32, (1, NB), 1).astype(jnp.float32) \
            * float(B_SORT)
        cnt = jnp.zeros((1, NB), jnp.float32)
        for e in range(N_ROUTED):
            cnt = cnt + (off_incl[:, e:e + 1] <= thr).astype(jnp.float32)
        bexp_ref[...] = jnp.minimum(cnt, float(N_ROUTED - 1)).astype(jnp.int32)


def _full(shape):
    return pl.BlockSpec(shape, lambda i: tuple(0 for _ in shape))


def _full2(shape):
    return pl.BlockSpec(shape, lambda i, b: tuple(0 for _ in shape))


def _tc1(flat, Wg, bg, interpret=False):
    grid = (N_TB,)
    return pl.pallas_call(
        _tc1_body,
        grid=grid,
        in_specs=[
            pl.BlockSpec((T_BLOCK, D_MODEL), lambda i: (i, 0)),
            _full(Wg.shape), _full(bg.shape),
        ],
        out_specs=[
            pl.BlockSpec((T_BLOCK, TOP_K), lambda i: (i, 0)),
            pl.BlockSpec((T_BLOCK, TOP_K), lambda i: (i, 0)),
            pl.BlockSpec((T_BLOCK, TOP_K), lambda i: (i, 0)),
            pl.BlockSpec((1, N_ROUTED), lambda i: (0, 0)),
            pl.BlockSpec((1, NB), lambda i: (0, 0)),
        ],
        out_shape=[
            jax.ShapeDtypeStruct((T_TOKENS, TOP_K), jnp.int32),
            jax.ShapeDtypeStruct((T_TOKENS, TOP_K), jnp.float32),
            jax.ShapeDtypeStruct((T_TOKENS, TOP_K), jnp.int32),
            jax.ShapeDtypeStruct((1, N_ROUTED), jnp.int32),
            jax.ShapeDtypeStruct((1, NB), jnp.int32),
        ],
        scratch_shapes=[pltpu.VMEM((1, N_ROUTED), jnp.float32)],
        interpret=interpret,
    )(flat, Wg, bg)


def _worker_id():
    return lax.axis_index("c") * (N_WORKER // 2) + lax.axis_index("s")


def _sc_move_body(eflat_hbm, rtot_hbm, basew_hbm, x_hbm,
                  xs_hbm, pos_hbm, e_buf, rtot_v, base_v, pos_v,
                  rows_a, rows_b, gsem, ssem):
    wid = _worker_id()
    iota = lax.broadcasted_iota(jnp.int32, (16,), 0)

    pltpu.sync_copy(eflat_hbm.at[pl.ds(wid * PW, PW)], e_buf)
    pltpu.sync_copy(rtot_hbm.at[pl.ds(wid * PW, PW)], rtot_v)
    pltpu.sync_copy(basew_hbm, base_v)

    poss = []
    for c in range(N_CHUNK):
        ev = e_buf[pl.ds(c * 16, 16)]
        rt = rtot_v[pl.ds(c * 16, 16)]
        posc = plsc.load_gather(base_v, [ev]) + rt
        pos_v[c, pl.ds(0, 16)] = posc
        poss.append(posc)
    pltpu.sync_copy(pos_v, pos_hbm.at[wid])

    bufs = (rows_a, rows_b)
    gd = [None] * N_CHUNK
    sd = [None] * N_CHUNK
    tok0 = (wid * PW + iota) >> 1
    gd[0] = pltpu.async_copy(x_hbm.at[tok0], bufs[0], gsem)
    for c in range(N_CHUNK):
        buf = bufs[c % 2]
        if c >= 1:
            sd[c - 1].wait()
        gd[c].wait()
        sd[c] = pltpu.async_copy(buf, xs_hbm.at[poss[c]], ssem)
        if c + 1 < N_CHUNK:
            tok = (wid * PW + (c + 1) * 16 + iota) >> 1
            gd[c + 1] = pltpu.async_copy(x_hbm.at[tok], bufs[(c + 1) % 2], gsem)
    sd[N_CHUNK - 1].wait()


def _sc_combine_body(pos_hbm, ew_hbm, sh_hbm, y_hbm, out_hbm,
                     posv, wv, yr_a, yr_b, sh_a, sh_b, outv,
                     ysem, hsem, osem):
    wid = _worker_id()
    iota = lax.broadcasted_iota(jnp.int32, (16,), 0)

    pltpu.sync_copy(pos_hbm.at[wid], posv)
    pltpu.sync_copy(ew_hbm.at[pl.ds(wid * PW, PW)], wv)

    ybufs = (yr_a, yr_b)
    sbufs = (sh_a, sh_b)
    gd = [None] * N_CHUNK
    hd = [None] * N_CHUNK
    od = [None] * N_CHUNK

    def _issue(c):
        pr = posv[c, pl.ds(0, 16)]
        gd[c] = pltpu.async_copy(y_hbm.at[pr], ybufs[c % 2], ysem)
        tb = wid * (PW // 2) + c * 8
        hd[c] = pltpu.async_copy(sh_hbm.at[pl.ds(tb, 8)], sbufs[c % 2], hsem)

    _issue(0)
    for c in range(N_CHUNK):
        yrows = ybufs[c % 2]
        shv = sbufs[c % 2]
        gd[c].wait()
        hd[c].wait()
        wreg = wv[pl.ds(c * 16, 16)]
        ws = []
        for j in range(8):
            ws.append(jnp.sum(jnp.where(iota == 2 * j, wreg, 0.0)))
            ws.append(jnp.sum(jnp.where(iota == 2 * j + 1, wreg, 0.0)))
        if c >= 1:
            od[c - 1].wait()

        def _col(col, _, ws=ws, yrows=yrows, shv=shv):
            s = pl.ds(col * 16, 16)
            for j in range(8):
                outv[j, s] = shv[j, s] + ws[2 * j] * yrows[2 * j, s] \
                    + ws[2 * j + 1] * yrows[2 * j + 1, s]
            return 0

        lax.fori_loop(0, D_MODEL // 16, _col, 0)
        tb = wid * (PW // 2) + c * 8
        od[c] = pltpu.async_copy(outv, out_hbm.at[pl.ds(tb, 8)], osem)
        if c + 1 < N_CHUNK:
            _issue(c + 1)
    od[N_CHUNK - 1].wait()


@functools.cache
def _sc_kernels():
    mesh = plsc.VectorSubcoreMesh(core_axis_name="c", subcore_axis_name="s")
    sc_params = pltpu.CompilerParams(needs_layout_passes=False)
    move = pl.kernel(
        _sc_move_body,
        out_type=(
            jax.ShapeDtypeStruct((L_PAD, D_MODEL), jnp.float32),
            jax.ShapeDtypeStruct((N_WORKER, N_CHUNK, 16), jnp.int32),
        ),
        mesh=mesh,
        compiler_params=sc_params,
        scratch_types=(
            pltpu.VMEM((PW,), jnp.int32),
            pltpu.VMEM((PW,), jnp.int32),
            pltpu.VMEM((16,), jnp.int32),
            pltpu.VMEM((N_CHUNK, 16), jnp.int32),
            pltpu.VMEM((16, D_MODEL), jnp.float32),
            pltpu.VMEM((16, D_MODEL), jnp.float32),
            pltpu.SemaphoreType.DMA,
            pltpu.SemaphoreType.DMA,
        ),
    )
    combine = pl.kernel(
        _sc_combine_body,
        out_type=jax.ShapeDtypeStruct((T_TOKENS, D_MODEL), jnp.float32),
        mesh=mesh,
        compiler_params=sc_params,
        scratch_types=(
            pltpu.VMEM((N_CHUNK, 16), jnp.int32),
            pltpu.VMEM((PW,), jnp.float32),
            pltpu.VMEM((16, D_MODEL), jnp.float32),
            pltpu.VMEM((16, D_MODEL), jnp.float32),
            pltpu.VMEM((8, D_MODEL), jnp.float32),
            pltpu.VMEM((8, D_MODEL), jnp.float32),
            pltpu.VMEM((8, D_MODEL), jnp.float32),
            pltpu.SemaphoreType.DMA,
            pltpu.SemaphoreType.DMA,
            pltpu.SemaphoreType.DMA,
        ),
    )
    return move, combine


def _gmm_body(bexp_ref, xs_ref, w1_ref, w2_ref, b1_ref, b2_ref, y_ref):
    be = bexp_ref[pl.program_id(0)]
    xb = xs_ref[...]
    h = jnp.maximum(
        jnp.dot(xb, w1_ref[be], preferred_element_type=jnp.float32)
        + b1_ref[be], 0.0)
    y_ref[...] = jnp.dot(h, w2_ref[be], preferred_element_type=jnp.float32) \
        + b2_ref[be]


def _gmm(bexp, xs, Wr1, Wr2, br1r, br2r, interpret=False):
    grid_spec = pltpu.PrefetchScalarGridSpec(
        num_scalar_prefetch=1,
        grid=(NB,),
        in_specs=[
            pl.BlockSpec((B_SORT, D_MODEL), lambda i, b: (i, 0)),
            _full2(Wr1.shape), _full2(Wr2.shape),
            _full2(br1r.shape), _full2(br2r.shape),
        ],
        out_specs=pl.BlockSpec((B_SORT, D_MODEL), lambda i, b: (i, 0)),
    )
    return pl.pallas_call(
        _gmm_body,
        grid_spec=grid_spec,
        out_shape=jax.ShapeDtypeStruct((L_PAD, D_MODEL), jnp.float32),
        interpret=interpret,
    )(bexp, xs, Wr1, Wr2, br1r, br2r)


def kernel(x, Ws1, bs1, Ws2, bs2, Wr1, br1, Wr2, br2, Wg, bg):
    B, S, D = x.shape
    flat = x.reshape(T_TOKENS, D_MODEL)
    eidx, ew, rtot, base, bexp = _tc1(flat, Wg, bg)
    shared = _tc_shared(flat, Ws1, bs1, Ws2, bs2)
    eflat = eidx.reshape(N_PAIRS)
    ewflat = ew.reshape(N_PAIRS)
    move, combine = _sc_kernels()
    xs, pos = move(eflat, rtot.reshape(N_PAIRS), base.reshape(N_ROUTED), flat)
    y = _gmm(bexp.reshape(NB), xs, Wr1, Wr2,
             br1.reshape(N_ROUTED, 1, D_HIDDEN),
             br2.reshape(N_ROUTED, 1, D_MODEL))
    out = combine(pos, ewflat, shared, y)
    return out.reshape(B, S, D)


# speedup vs baseline: 1.2318x; 1.0715x over previous
import functools

import jax
import jax.numpy as jnp
from jax import lax
from jax.experimental import pallas as pl
from jax.experimental.pallas import tpu as pltpu
from jax.experimental.pallas import tpu_sc as plsc

D_MODEL = 1024
D_HIDDEN = 256
N_SHARED = 2
N_ROUTED = 16
TOP_K = 2
T_TOKENS = 2048
N_PAIRS = T_TOKENS * TOP_K
B_SORT = 256
L_PAD = N_PAIRS + N_ROUTED * B_SORT
NB = L_PAD // B_SORT
T_BLOCK = 256
N_TB = T_TOKENS // T_BLOCK

N_WORKER = 32
PW = N_PAIRS // N_WORKER
N_CHUNK = PW // 16


def _tc_shared_body(x_ref, Ws1_ref, bs1_ref, Ws2_ref, bs2_ref, sh_ref):
    xb = x_ref[...]
    xb16 = xb.astype(jnp.bfloat16)
    acc = jnp.zeros_like(xb)
    for s in range(N_SHARED):
        h = jnp.maximum(
            jnp.dot(xb16, Ws1_ref[s].astype(jnp.bfloat16),
                    preferred_element_type=jnp.float32)
            + bs1_ref[s][None, :], 0.0)
        acc = acc + jnp.dot(h.astype(jnp.bfloat16),
                            Ws2_ref[s].astype(jnp.bfloat16),
                            preferred_element_type=jnp.float32) \
            + bs2_ref[s][None, :]
    sh_ref[...] = acc * (1.0 / N_SHARED)


def _tc_shared(flat, Ws1, bs1, Ws2, bs2, interpret=False):
    return pl.pallas_call(
        _tc_shared_body,
        grid=(N_TB,),
        in_specs=[
            pl.BlockSpec((T_BLOCK, D_MODEL), lambda i: (i, 0)),
            _full(Ws1.shape), _full(bs1.shape),
            _full(Ws2.shape), _full(bs2.shape),
        ],
        out_specs=pl.BlockSpec((T_BLOCK, D_MODEL), lambda i: (i, 0)),
        out_shape=jax.ShapeDtypeStruct((T_TOKENS, D_MODEL), jnp.float32),
        interpret=interpret,
    )(flat, Ws1, bs1, Ws2, bs2)


def _tc1_body(x_ref, Wg_ref, bg_ref,
              eidx_ref, ew_ref, rtot_ref, base_ref, bexp_ref,
              carry_ref):
    i = pl.program_id(0)
    xb = x_ref[...]
    logits = jnp.dot(xb, Wg_ref[...], preferred_element_type=jnp.float32) \
        + bg_ref[...][None, :]
    iota = lax.broadcasted_iota(jnp.int32, logits.shape, 1)
    m1 = jnp.max(logits, axis=-1, keepdims=True)
    i1 = jnp.min(jnp.where(logits == m1, iota, N_ROUTED), axis=-1, keepdims=True)
    masked = jnp.where(iota == i1, -jnp.inf, logits)
    m2 = jnp.max(masked, axis=-1, keepdims=True)
    i2 = jnp.min(jnp.where(masked == m2, iota, N_ROUTED), axis=-1, keepdims=True)
    w1 = 1.0 / (1.0 + jnp.exp(m2 - m1))
    eidx_ref[...] = jnp.concatenate([i1, i2], axis=1)
    ew_ref[...] = jnp.concatenate([w1, 1.0 - w1], axis=1)

    @pl.when(i == 0)
    def _():
        carry_ref[...] = jnp.zeros((1, N_ROUTED), jnp.float32)

    m0 = (iota == i1).astype(jnp.float32)
    m1h = (iota == i2).astype(jnp.float32)
    r = lax.broadcasted_iota(jnp.int32, (T_BLOCK, T_BLOCK), 0)
    c = lax.broadcasted_iota(jnp.int32, (T_BLOCK, T_BLOCK), 1)
    tril = (r > c).astype(jnp.float32)
    c0 = jnp.dot(tril, m0, preferred_element_type=jnp.float32)
    c1 = jnp.dot(tril, m1h, preferred_element_type=jnp.float32)
    carry = carry_ref[...]
    both = c0 + c1 + carry
    r0 = jnp.sum(both * m0, axis=1, keepdims=True)
    r1 = jnp.sum(both * m1h, axis=1, keepdims=True)
    rtot_ref[...] = jnp.concatenate([r0, r1], axis=1).astype(jnp.int32)
    new_carry = carry + jnp.sum(m0 + m1h, axis=0, keepdims=True)
    carry_ref[...] = new_carry

    @pl.when(i == N_TB - 1)
    def _():
        n = new_carry
        padded = jnp.floor((n + (B_SORT - 1)) * (1.0 / B_SORT)) * B_SORT
        e_r = lax.broadcasted_iota(jnp.int32, (N_ROUTED, N_ROUTED), 0)
        e_c = lax.broadcasted_iota(jnp.int32, (N_ROUTED, N_ROUTED), 1)
        incl = (e_r <= e_c).astype(jnp.float32)
        off_incl = jnp.dot(padded, incl, preferred_element_type=jnp.float32)
        base_ref[...] = (off_incl - padded).astype(jnp.int32)
        thr = lax.broadcasted_iota(jnp.int32, (1, NB), 1).astype(jnp.float32) \
            * float(B_SORT)
        cnt = jnp.zeros((1, NB), jnp.float32)
        for e in range(N_ROUTED):
            cnt = cnt + (off_incl[:, e:e + 1] <= thr).astype(jnp.float32)
        bexp_ref[...] = jnp.minimum(cnt, float(N_ROUTED - 1)).astype(jnp.int32)


def _full(shape):
    return pl.BlockSpec(shape, lambda i: tuple(0 for _ in shape))


def _full2(shape):
    return pl.BlockSpec(shape, lambda i, b: tuple(0 for _ in shape))


def _tc1(flat, Wg, bg, interpret=False):
    grid = (N_TB,)
    return pl.pallas_call(
        _tc1_body,
        grid=grid,
        in_specs=[
            pl.BlockSpec((T_BLOCK, D_MODEL), lambda i: (i, 0)),
            _full(Wg.shape), _full(bg.shape),
        ],
        out_specs=[
            pl.BlockSpec((T_BLOCK, TOP_K), lambda i: (i, 0)),
            pl.BlockSpec((T_BLOCK, TOP_K), lambda i: (i, 0)),
            pl.BlockSpec((T_BLOCK, TOP_K), lambda i: (i, 0)),
            pl.BlockSpec((1, N_ROUTED), lambda i: (0, 0)),
            pl.BlockSpec((1, NB), lambda i: (0, 0)),
        ],
        out_shape=[
            jax.ShapeDtypeStruct((T_TOKENS, TOP_K), jnp.int32),
            jax.ShapeDtypeStruct((T_TOKENS, TOP_K), jnp.float32),
            jax.ShapeDtypeStruct((T_TOKENS, TOP_K), jnp.int32),
            jax.ShapeDtypeStruct((1, N_ROUTED), jnp.int32),
            jax.ShapeDtypeStruct((1, NB), jnp.int32),
        ],
        scratch_shapes=[pltpu.VMEM((1, N_ROUTED), jnp.float32)],
        interpret=interpret,
    )(flat, Wg, bg)


def _worker_id():
    return lax.axis_index("c") * (N_WORKER // 2) + lax.axis_index("s")


def _sc_move_body(eflat_hbm, rtot_hbm, basew_hbm, x_hbm,
                  xs_hbm, pos_hbm, e_buf, rtot_v, base_v, pos_v,
                  rows_a, rows_b, gsem, ssem):
    wid = _worker_id()
    iota = lax.broadcasted_iota(jnp.int32, (16,), 0)

    pltpu.sync_copy(eflat_hbm.at[pl.ds(wid * PW, PW)], e_buf)
    pltpu.sync_copy(rtot_hbm.at[pl.ds(wid * PW, PW)], rtot_v)
    pltpu.sync_copy(basew_hbm, base_v)

    poss = []
    for c in range(N_CHUNK):
        ev = e_buf[pl.ds(c * 16, 16)]
        rt = rtot_v[pl.ds(c * 16, 16)]
        posc = plsc.load_gather(base_v, [ev]) + rt
        pos_v[c, pl.ds(0, 16)] = posc
        poss.append(posc)
    pltpu.sync_copy(pos_v, pos_hbm.at[wid])

    bufs = (rows_a, rows_b)
    gd = [None] * N_CHUNK
    sd = [None] * N_CHUNK
    tok0 = (wid * PW + iota) >> 1
    gd[0] = pltpu.async_copy(x_hbm.at[tok0], bufs[0], gsem)
    for c in range(N_CHUNK):
        buf = bufs[c % 2]
        if c >= 1:
            sd[c - 1].wait()
        gd[c].wait()
        sd[c] = pltpu.async_copy(buf, xs_hbm.at[poss[c]], ssem)
        if c + 1 < N_CHUNK:
            tok = (wid * PW + (c + 1) * 16 + iota) >> 1
            gd[c + 1] = pltpu.async_copy(x_hbm.at[tok], bufs[(c + 1) % 2], gsem)
    sd[N_CHUNK - 1].wait()


def _sc_combine_body(pos_hbm, ew_hbm, sh_hbm, y_hbm, out_hbm,
                     posv, wv, yr_a, yr_b, yr_c, sh_a, sh_b, sh_c, outv,
                     ysem, hsem, osem):
    wid = _worker_id()
    iota = lax.broadcasted_iota(jnp.int32, (16,), 0)

    pltpu.sync_copy(pos_hbm.at[wid], posv)
    pltpu.sync_copy(ew_hbm.at[pl.ds(wid * PW, PW)], wv)

    ybufs = (yr_a, yr_b, yr_c)
    sbufs = (sh_a, sh_b, sh_c)
    nbuf = 3
    gd = [None] * N_CHUNK
    hd = [None] * N_CHUNK
    od = [None] * N_CHUNK

    def _issue(c):
        pr = posv[c, pl.ds(0, 16)]
        gd[c] = pltpu.async_copy(y_hbm.at[pr], ybufs[c % nbuf], ysem)
        tb = wid * (PW // 2) + c * 8
        hd[c] = pltpu.async_copy(sh_hbm.at[pl.ds(tb, 8)], sbufs[c % nbuf], hsem)

    _issue(0)
    _issue(1)
    for c in range(N_CHUNK):
        yrows = ybufs[c % nbuf]
        shv = sbufs[c % nbuf]
        gd[c].wait()
        hd[c].wait()
        wreg = wv[pl.ds(c * 16, 16)]
        ws = []
        for j in range(8):
            ws.append(jnp.sum(jnp.where(iota == 2 * j, wreg, 0.0)))
            ws.append(jnp.sum(jnp.where(iota == 2 * j + 1, wreg, 0.0)))
        if c >= 1:
            od[c - 1].wait()

        def _col(col, _, ws=ws, yrows=yrows, shv=shv):
            s = pl.ds(col * 16, 16)
            for j in range(8):
                outv[j, s] = shv[j, s] + ws[2 * j] * yrows[2 * j, s] \
                    + ws[2 * j + 1] * yrows[2 * j + 1, s]
            return 0

        lax.fori_loop(0, D_MODEL // 16, _col, 0)
        tb = wid * (PW // 2) + c * 8
        od[c] = pltpu.async_copy(outv, out_hbm.at[pl.ds(tb, 8)], osem)
        if c + 2 < N_CHUNK:
            _issue(c + 2)
    od[N_CHUNK - 1].wait()


@functools.cache
def _sc_kernels():
    mesh = plsc.VectorSubcoreMesh(core_axis_name="c", subcore_axis_name="s")
    sc_params = pltpu.CompilerParams(needs_layout_passes=False)
    move = pl.kernel(
        _sc_move_body,
        out_type=(
            jax.ShapeDtypeStruct((L_PAD, D_MODEL), jnp.float32),
            jax.ShapeDtypeStruct((N_WORKER, N_CHUNK, 16), jnp.int32),
        ),
        mesh=mesh,
        compiler_params=sc_params,
        scratch_types=(
            pltpu.VMEM((PW,), jnp.int32),
            pltpu.VMEM((PW,), jnp.int32),
            pltpu.VMEM((16,), jnp.int32),
            pltpu.VMEM((N_CHUNK, 16), jnp.int32),
            pltpu.VMEM((16, D_MODEL), jnp.float32),
            pltpu.VMEM((16, D_MODEL), jnp.float32),
            pltpu.SemaphoreType.DMA,
            pltpu.SemaphoreType.DMA,
        ),
    )
    combine = pl.kernel(
        _sc_combine_body,
        out_type=jax.ShapeDtypeStruct((T_TOKENS, D_MODEL), jnp.float32),
        mesh=mesh,
        compiler_params=sc_params,
        scratch_types=(
            pltpu.VMEM((N_CHUNK, 16), jnp.int32),
            pltpu.VMEM((PW,), jnp.float32),
            pltpu.VMEM((16, D_MODEL), jnp.float32),
            pltpu.VMEM((16, D_MODEL), jnp.float32),
            pltpu.VMEM((16, D_MODEL), jnp.float32),
            pltpu.VMEM((8, D_MODEL), jnp.float32),
            pltpu.VMEM((8, D_MODEL), jnp.float32),
            pltpu.VMEM((8, D_MODEL), jnp.float32),
            pltpu.VMEM((8, D_MODEL), jnp.float32),
            pltpu.SemaphoreType.DMA,
            pltpu.SemaphoreType.DMA,
            pltpu.SemaphoreType.DMA,
        ),
    )
    return move, combine


def _gmm_body(bexp_ref, xs_ref, w1_ref, w2_ref, b1_ref, b2_ref, y_ref):
    be = bexp_ref[pl.program_id(0)]
    xb = xs_ref[...]
    h = jnp.maximum(
        jnp.dot(xb, w1_ref[be], preferred_element_type=jnp.float32)
        + b1_ref[be], 0.0)
    y_ref[...] = jnp.dot(h, w2_ref[be], preferred_element_type=jnp.float32) \
        + b2_ref[be]


def _gmm(bexp, xs, Wr1, Wr2, br1r, br2r, interpret=False):
    grid_spec = pltpu.PrefetchScalarGridSpec(
        num_scalar_prefetch=1,
        grid=(NB,),
        in_specs=[
            pl.BlockSpec((B_SORT, D_MODEL), lambda i, b: (i, 0)),
            _full2(Wr1.shape), _full2(Wr2.shape),
            _full2(br1r.shape), _full2(br2r.shape),
        ],
        out_specs=pl.BlockSpec((B_SORT, D_MODEL), lambda i, b: (i, 0)),
    )
    return pl.pallas_call(
        _gmm_body,
        grid_spec=grid_spec,
        out_shape=jax.ShapeDtypeStruct((L_PAD, D_MODEL), jnp.float32),
        interpret=interpret,
    )(bexp, xs, Wr1, Wr2, br1r, br2r)


def kernel(x, Ws1, bs1, Ws2, bs2, Wr1, br1, Wr2, br2, Wg, bg):
    B, S, D = x.shape
    flat = x.reshape(T_TOKENS, D_MODEL)
    eidx, ew, rtot, base, bexp = _tc1(flat, Wg, bg)
    shared = _tc_shared(flat, Ws1, bs1, Ws2, bs2)
    eflat = eidx.reshape(N_PAIRS)
    ewflat = ew.reshape(N_PAIRS)
    move, combine = _sc_kernels()
    xs, pos = move(eflat, rtot.reshape(N_PAIRS), base.reshape(N_ROUTED), flat)
    y = _gmm(bexp.reshape(NB), xs, Wr1, Wr2,
             br1.reshape(N_ROUTED, 1, D_HIDDEN),
             br2.reshape(N_ROUTED, 1, D_MODEL))
    out = combine(pos, ewflat, shared, y)
    return out.reshape(B, S, D)
